# baseline (speedup 1.0000x reference)
import functools

import jax
import jax.numpy as jnp
import numpy as np
from jax import lax
from jax.experimental import pallas as pl
from jax.experimental.pallas import tpu as pltpu
from jax.experimental.pallas import tpu_sc as plsc

D_MODEL = 1024
D_FF = 2048
E = 8
S = 2048
CAP = S // E
EPS = float(np.finfo(float).eps)
F32_INF_BITS = 0x7F800000

NC = 2
NS = 16
NW = NC * NS

_sc_params = pltpu.CompilerParams(needs_layout_passes=False)


def _gating_body(x_ref, wg_ref, ru_ref, gates_ref, ci0_ref, cw0_ref,
                 ci1_ref, cw1_ref, loss_ref):
    x = x_ref[...]
    logits = jnp.dot(x, wg_ref[...], preferred_element_type=jnp.float32)
    m = jnp.max(logits, axis=1, keepdims=True)
    ex = jnp.exp(logits - m)
    g = ex / jnp.sum(ex, axis=1, keepdims=True)

    cols = jax.lax.broadcasted_iota(jnp.int32, (S, E), 1)
    m1 = jnp.max(g, axis=1, keepdims=True)
    e1 = jnp.min(jnp.where(g == m1, cols, E), axis=1, keepdims=True)
    g_m = jnp.where(cols == e1, -jnp.inf, g)
    m2 = jnp.max(g_m, axis=1, keepdims=True)
    e2 = jnp.min(jnp.where(g_m == m2, cols, E), axis=1, keepdims=True)
    s12 = m1 + m2
    g1 = m1 / s12
    g2 = m2 / s12

    mean_g = jnp.mean(g, axis=0, keepdims=True)
    counts = jnp.sum((cols == e1).astype(jnp.float32), axis=0, keepdims=True)
    loss_ref[0, 0] = jnp.sum(counts / S * mean_g) / E * 0.1

    gcw1 = jnp.where(cols == e1, g1, 0.0)
    gcw2 = jnp.where(cols == e2, g2, 0.0)
    gcw2 = jnp.where(gcw2 > ru_ref[...], gcw2, 0.0)
    gcw = gcw1 + gcw2

    def bs_body(_, lohi):
        lo, hi = lohi
        mid = lo + (hi - lo) // 2
        t = jax.lax.bitcast_convert_type(mid, jnp.float32)
        cnt = jnp.sum((gcw > t).astype(jnp.int32), axis=0, keepdims=True)
        pred = cnt < CAP
        return jnp.where(pred, lo, mid + 1), jnp.where(pred, mid, hi)

    lo0 = jnp.zeros((1, E), jnp.int32)
    hi0 = jnp.full((1, E), F32_INF_BITS, jnp.int32)
    lo, _ = jax.lax.fori_loop(0, 31, bs_body, (lo0, hi0))
    t = jax.lax.bitcast_convert_type(lo, jnp.float32)

    n_gt = jnp.sum((gcw > t).astype(jnp.int32), axis=0, keepdims=True)
    quota = (CAP - n_gt).astype(jnp.float32)
    eq = (gcw == t) & (gcw > 0.0)
    eqf = eq.astype(jnp.float32)
    ri = jax.lax.broadcasted_iota(jnp.int32, (S, S), 0)
    ci = jax.lax.broadcasted_iota(jnp.int32, (S, S), 1)
    tri = (ci < ri).astype(jnp.float32)
    eq_prefix = jnp.dot(tri, eqf, preferred_element_type=jnp.float32)
    sel = (gcw > t) | (eq & (eq_prefix < quota))
    gates_m = jnp.where(sel, gcw, 0.0)
    gates_ref[...] = gates_m

    slots = jnp.dot(tri, sel.astype(jnp.float32),
                    preferred_element_type=jnp.float32).astype(jnp.int32)
    flatpos = cols * CAP + slots
    ci0_ref[...] = jnp.sum(jnp.where((cols == e1) & sel, flatpos, 0),
                           axis=1, keepdims=True)
    cw0_ref[...] = jnp.sum(jnp.where(cols == e1, gates_m, 0.0),
                           axis=1, keepdims=True)
    ci1_ref[...] = jnp.sum(jnp.where((cols == e2) & sel, flatpos, 0),
                           axis=1, keepdims=True)
    cw1_ref[...] = jnp.sum(jnp.where(cols == e2, gates_m, 0.0),
                           axis=1, keepdims=True)


def _ffn_body(xd_ref, w1_ref, b1_ref, w2_ref, b2_ref, o_ref):
    h = jnp.dot(xd_ref[...], w1_ref[0], preferred_element_type=jnp.float32)
    h = jnp.maximum(h + b1_ref[0], 0.0)
    o_ref[...] = (jnp.dot(h, w2_ref[0], preferred_element_type=jnp.float32)
                  + b2_ref[0])


def _sc_route_body(gates_hbm, x_hbm, xd_hbm, gv, idxb, xbuf, sem):
    wid = lax.axis_index("s") * NC + lax.axis_index("c")

    @pl.when(wid < E)
    def _():
        pltpu.sync_copy(gates_hbm, gv)
        z = jnp.zeros((16,), jnp.int32)
        for zz in range((CAP + 16) // 16):
            idxb[pl.ds(zz * 16, 16)] = z
        ecol = jnp.full((16,), wid, jnp.int32)

        def step(i, cnt):
            rows = lax.iota(jnp.int32, 16) + i * 16
            vals = plsc.load_gather(gv, [rows * E + ecol])
            msk = vals > 0.0
            plsc.store_compressed(idxb.at[pl.ds(cnt, 16)], rows, mask=msk)
            return cnt + jnp.max(plsc.all_reduce_population_count(msk))

        lax.fori_loop(0, S // 16, step, jnp.int32(0))
        for ch in range(CAP // 64):
            pltpu.async_copy(
                x_hbm.at[idxb.at[pl.ds(ch * 64, 64)]], xbuf, sem).wait()
            pltpu.sync_copy(
                xbuf, xd_hbm.at[pl.ds(wid * CAP + ch * 64, 64)])


_CH = 32


def _sc_combine_body(o_hbm, ci0_hbm, cw0_hbm, ci1_hbm, cw1_hbm, y_hbm,
                     i0v, i1v, w0v, w1v, b0, b1, yb, sem):
    wid = lax.axis_index("s") * NC + lax.axis_index("c")
    per_w = S // NW
    base = wid * per_w
    for c in range(per_w // _CH):
        tb = base + c * _CH
        pltpu.sync_copy(ci0_hbm.at[pl.ds(tb, _CH)], i0v)
        pltpu.sync_copy(ci1_hbm.at[pl.ds(tb, _CH)], i1v)
        pltpu.sync_copy(cw0_hbm.at[pl.ds(tb, _CH)], w0v)
        pltpu.sync_copy(cw1_hbm.at[pl.ds(tb, _CH)], w1v)
        pltpu.async_copy(o_hbm.at[i0v], b0, sem).wait()
        pltpu.async_copy(o_hbm.at[i1v], b1, sem).wait()

        def tok(t, carry):
            grp = (t // 16) * 16
            lane = jnp.full((16,), t % 16, jnp.int32)
            w0 = w0v[pl.ds(grp, 16)].at[lane].get(mode="promise_in_bounds")
            w1 = w1v[pl.ds(grp, 16)].at[lane].get(mode="promise_in_bounds")
            for j in range(D_MODEL // 16):
                v = (b0[t, pl.ds(j * 16, 16)] * w0
                     + b1[t, pl.ds(j * 16, 16)] * w1)
                v = jnp.where(v == 0.0, jnp.float32(EPS), v)
                yb[t, pl.ds(j * 16, 16)] = v
            return carry

        lax.fori_loop(0, _CH, tok, 0)
        pltpu.sync_copy(yb, y_hbm.at[pl.ds(tb, _CH)])


_SC_KERNELS = None


def _get_sc_kernels():
    global _SC_KERNELS
    if _SC_KERNELS is None:
        mesh = plsc.VectorSubcoreMesh(core_axis_name="c", subcore_axis_name="s",
                                      num_cores=NC, num_subcores=NS)
        route = pl.kernel(
            _sc_route_body, mesh=mesh, compiler_params=_sc_params,
            out_type=jax.ShapeDtypeStruct((S, D_MODEL), jnp.float32),
            scratch_types=[
                pltpu.VMEM((S * E,), jnp.float32),
                pltpu.VMEM((CAP + 16,), jnp.int32),
                pltpu.VMEM((64, D_MODEL), jnp.float32),
                pltpu.SemaphoreType.DMA,
            ],
        )
        combine = pl.kernel(
            _sc_combine_body, mesh=mesh, compiler_params=_sc_params,
            out_type=jax.ShapeDtypeStruct((S, D_MODEL), jnp.float32),
            scratch_types=[
                pltpu.VMEM((_CH,), jnp.int32),
                pltpu.VMEM((_CH,), jnp.int32),
                pltpu.VMEM((_CH,), jnp.float32),
                pltpu.VMEM((_CH,), jnp.float32),
                pltpu.VMEM((_CH, D_MODEL), jnp.float32),
                pltpu.VMEM((_CH, D_MODEL), jnp.float32),
                pltpu.VMEM((_CH, D_MODEL), jnp.float32),
                pltpu.SemaphoreType.DMA,
            ],
        )
        _SC_KERNELS = (route, combine)
    return _SC_KERNELS


def kernel(x, w_gate, W1, b1, W2, b2):
    ru = jax.random.uniform(jax.random.key(42), (S, E), dtype=jnp.float32) / 2.0
    sc_route, sc_combine = _get_sc_kernels()

    gates, ci0, cw0, ci1, cw1, loss = pl.pallas_call(
        _gating_body,
        out_shape=(
            jax.ShapeDtypeStruct((S, E), jnp.float32),
            jax.ShapeDtypeStruct((S, 1), jnp.int32),
            jax.ShapeDtypeStruct((S, 1), jnp.float32),
            jax.ShapeDtypeStruct((S, 1), jnp.int32),
            jax.ShapeDtypeStruct((S, 1), jnp.float32),
            jax.ShapeDtypeStruct((1, 1), jnp.float32),
        ),
        in_specs=[
            pl.BlockSpec((S, D_MODEL), lambda: (0, 0)),
            pl.BlockSpec((D_MODEL, E), lambda: (0, 0)),
            pl.BlockSpec((S, E), lambda: (0, 0)),
        ],
        out_specs=(
            pl.BlockSpec((S, E), lambda: (0, 0)),
            pl.BlockSpec((S, 1), lambda: (0, 0)),
            pl.BlockSpec((S, 1), lambda: (0, 0)),
            pl.BlockSpec((S, 1), lambda: (0, 0)),
            pl.BlockSpec((S, 1), lambda: (0, 0)),
            pl.BlockSpec((1, 1), lambda: (0, 0), memory_space=pltpu.SMEM),
        ),
    )(x, w_gate, ru)

    xd = sc_route(gates.reshape(S * E), x)

    o = pl.pallas_call(
        _ffn_body,
        grid=(E,),
        out_shape=jax.ShapeDtypeStruct((S, D_MODEL), jnp.float32),
        in_specs=[
            pl.BlockSpec((CAP, D_MODEL), lambda e: (e, 0)),
            pl.BlockSpec((1, D_MODEL, D_FF), lambda e: (e, 0, 0)),
            pl.BlockSpec((1, 1, D_FF), lambda e: (e, 0, 0)),
            pl.BlockSpec((1, D_FF, D_MODEL), lambda e: (e, 0, 0)),
            pl.BlockSpec((1, 1, D_MODEL), lambda e: (e, 0, 0)),
        ],
        out_specs=pl.BlockSpec((CAP, D_MODEL), lambda e: (e, 0)),
    )(xd, W1, b1.reshape(E, 1, D_FF), W2, b2.reshape(E, 1, D_MODEL))

    y = sc_combine(o, ci0.reshape(S), cw0.reshape(S),
                   ci1.reshape(S), cw1.reshape(S))

    return y, loss.reshape(())


# speedup vs baseline: 1.0744x; 1.0744x over previous
import functools

import jax
import jax.numpy as jnp
import numpy as np
from jax import lax
from jax.experimental import pallas as pl
from jax.experimental.pallas import tpu as pltpu
from jax.experimental.pallas import tpu_sc as plsc

D_MODEL = 1024
D_FF = 2048
E = 8
S = 2048
CAP = S // E
EPS = float(np.finfo(float).eps)
F32_INF_BITS = 0x7F800000

NC = 2
NS = 16
NW = NC * NS

_sc_params = pltpu.CompilerParams(needs_layout_passes=False)


def _gating_body(x_ref, wg_ref, ru_ref, gates_ref, ci0_ref, cw0_ref,
                 ci1_ref, cw1_ref, loss_ref):
    x = x_ref[...]
    logits = jnp.dot(x, wg_ref[...], preferred_element_type=jnp.float32)
    m = jnp.max(logits, axis=1, keepdims=True)
    ex = jnp.exp(logits - m)
    g = ex / jnp.sum(ex, axis=1, keepdims=True)

    cols = jax.lax.broadcasted_iota(jnp.int32, (S, E), 1)
    m1 = jnp.max(g, axis=1, keepdims=True)
    e1 = jnp.min(jnp.where(g == m1, cols, E), axis=1, keepdims=True)
    g_m = jnp.where(cols == e1, -jnp.inf, g)
    m2 = jnp.max(g_m, axis=1, keepdims=True)
    e2 = jnp.min(jnp.where(g_m == m2, cols, E), axis=1, keepdims=True)
    s12 = m1 + m2
    g1 = m1 / s12
    g2 = m2 / s12

    mean_g = jnp.mean(g, axis=0, keepdims=True)
    counts = jnp.sum((cols == e1).astype(jnp.float32), axis=0, keepdims=True)
    loss_ref[0, 0] = jnp.sum(counts / S * mean_g) / E * 0.1

    gcw1 = jnp.where(cols == e1, g1, 0.0)
    gcw2 = jnp.where(cols == e2, g2, 0.0)
    gcw2 = jnp.where(gcw2 > ru_ref[...], gcw2, 0.0)
    gcw = gcw1 + gcw2

    def bs_body(_, lohi):
        lo, hi = lohi
        mid = lo + (hi - lo) // 2
        t = jax.lax.bitcast_convert_type(mid, jnp.float32)
        cnt = jnp.sum((gcw > t).astype(jnp.int32), axis=0, keepdims=True)
        pred = cnt < CAP
        return jnp.where(pred, lo, mid + 1), jnp.where(pred, mid, hi)

    lo0 = jnp.zeros((1, E), jnp.int32)
    hi0 = jnp.full((1, E), F32_INF_BITS, jnp.int32)
    lo, _ = jax.lax.fori_loop(0, 31, bs_body, (lo0, hi0))
    t = jax.lax.bitcast_convert_type(lo, jnp.float32)

    gt = gcw > t
    n_gt = jnp.sum(gt.astype(jnp.int32), axis=0, keepdims=True)
    quota = (CAP - n_gt).astype(jnp.float32)
    eq = (gcw == t) & (gcw > 0.0)
    ri = jax.lax.broadcasted_iota(jnp.int32, (S, S), 0)
    ci = jax.lax.broadcasted_iota(jnp.int32, (S, S), 1)
    tri = (ci < ri).astype(jnp.float32)
    both = jnp.concatenate(
        [gt.astype(jnp.float32), eq.astype(jnp.float32)], axis=1)
    pref = jnp.dot(tri, both, preferred_element_type=jnp.float32)
    gt_prefix = pref[:, :E]
    eq_prefix = pref[:, E:]
    sel = gt | (eq & (eq_prefix < quota))
    gates_m = jnp.where(sel, gcw, 0.0)
    gates_ref[...] = gates_m

    slots = (gt_prefix + jnp.minimum(eq_prefix, quota)).astype(jnp.int32)
    flatpos = cols * CAP + slots
    ci0_ref[...] = jnp.sum(jnp.where((cols == e1) & sel, flatpos, 0),
                           axis=1, keepdims=True)
    cw0_ref[...] = jnp.sum(jnp.where(cols == e1, gates_m, 0.0),
                           axis=1, keepdims=True)
    ci1_ref[...] = jnp.sum(jnp.where((cols == e2) & sel, flatpos, 0),
                           axis=1, keepdims=True)
    cw1_ref[...] = jnp.sum(jnp.where(cols == e2, gates_m, 0.0),
                           axis=1, keepdims=True)


def _ffn_body(xd_ref, w1_ref, b1_ref, w2_ref, b2_ref, o_ref):
    h = jnp.dot(xd_ref[...], w1_ref[0], preferred_element_type=jnp.float32)
    h = jnp.maximum(h + b1_ref[0], 0.0)
    o_ref[...] = (jnp.dot(h, w2_ref[0], preferred_element_type=jnp.float32)
                  + b2_ref[0])


_RP = 4
_RROWS = CAP // _RP
_RCH = 32


def _sc_route_body(gates_hbm, x_hbm, xd_hbm, gv, idxb,
                   xb0, xb1, gs0, gs1, os0, os1):
    wid = lax.axis_index("s") * NC + lax.axis_index("c")
    e = wid // _RP
    part = wid - e * _RP

    pltpu.sync_copy(gates_hbm, gv)
    z = jnp.zeros((16,), jnp.int32)
    for zz in range((CAP + 16) // 16):
        idxb[pl.ds(zz * 16, 16)] = z
    ecol = jnp.full((16,), e, jnp.int32)

    def step(i, cnt):
        rows = lax.iota(jnp.int32, 16) + i * 16
        vals = plsc.load_gather(gv, [rows * E + ecol])
        msk = vals > 0.0
        plsc.store_compressed(idxb.at[pl.ds(cnt, 16)], rows, mask=msk)
        return cnt + jnp.max(plsc.all_reduce_population_count(msk))

    lax.fori_loop(0, S // 16, step, jnp.int32(0))

    base = part * _RROWS
    xbufs = (xb0, xb1)
    gsems = (gs0, gs1)
    osems = (os0, os1)
    nch = _RROWS // _RCH
    gh = [None] * nch
    oh = [None] * nch
    gh[0] = pltpu.async_copy(
        x_hbm.at[idxb.at[pl.ds(base, _RCH)]], xbufs[0], gsems[0])
    for c in range(nch):
        if c + 1 < nch:
            gh[c + 1] = pltpu.async_copy(
                x_hbm.at[idxb.at[pl.ds(base + (c + 1) * _RCH, _RCH)]],
                xbufs[(c + 1) % 2], gsems[(c + 1) % 2])
        gh[c].wait()
        oh[c] = pltpu.async_copy(
            xbufs[c % 2],
            xd_hbm.at[pl.ds(e * CAP + base + c * _RCH, _RCH)],
            osems[c % 2])
    for c in range(nch):
        oh[c].wait()


_CH = 16
_CPW = S // NW


def _sc_combine_body(o_hbm, ci0_hbm, cw0_hbm, ci1_hbm, cw1_hbm, y_hbm,
                     i0v, i1v, w0v, w1v, b0a, b0b, b1a, b1b, yba, ybb,
                     isem, gs0, gs1, os0, os1):
    wid = lax.axis_index("s") * NC + lax.axis_index("c")
    base = wid * _CPW

    h1 = pltpu.async_copy(ci0_hbm.at[pl.ds(base, _CPW)], i0v, isem)
    h2 = pltpu.async_copy(ci1_hbm.at[pl.ds(base, _CPW)], i1v, isem)
    h3 = pltpu.async_copy(cw0_hbm.at[pl.ds(base, _CPW)], w0v, isem)
    h4 = pltpu.async_copy(cw1_hbm.at[pl.ds(base, _CPW)], w1v, isem)
    h1.wait()
    h2.wait()
    h3.wait()
    h4.wait()

    b0s = (b0a, b0b)
    b1s = (b1a, b1b)
    ybs = (yba, ybb)
    gsems = (gs0, gs1)
    osems = (os0, os1)
    nch = _CPW // _CH
    gh = [None] * nch
    oh = [None] * nch

    def fire(c):
        gh[c] = (
            pltpu.async_copy(
                o_hbm.at[i0v.at[pl.ds(c * _CH, _CH)]], b0s[c % 2],
                gsems[c % 2]),
            pltpu.async_copy(
                o_hbm.at[i1v.at[pl.ds(c * _CH, _CH)]], b1s[c % 2],
                gsems[c % 2]),
        )

    fire(0)
    for c in range(nch):
        gh[c][0].wait()
        gh[c][1].wait()
        if c + 1 < nch:
            fire(c + 1)
        if c >= 2:
            oh[c - 2].wait()
        b0 = b0s[c % 2]
        b1 = b1s[c % 2]
        yb = ybs[c % 2]
        w0c = w0v[pl.ds(c * _CH, 16)]
        w1c = w1v[pl.ds(c * _CH, 16)]

        def tok(t, carry):
            lane = jnp.full((16,), t, jnp.int32)
            w0 = w0c.at[lane].get(mode="promise_in_bounds")
            w1 = w1c.at[lane].get(mode="promise_in_bounds")
            for j in range(D_MODEL // 16):
                v = (b0[t, pl.ds(j * 16, 16)] * w0
                     + b1[t, pl.ds(j * 16, 16)] * w1)
                v = jnp.where(v == 0.0, jnp.float32(EPS), v)
                yb[t, pl.ds(j * 16, 16)] = v
            return carry

        lax.fori_loop(0, _CH, tok, 0)
        oh[c] = pltpu.async_copy(
            yb, y_hbm.at[pl.ds(base + c * _CH, _CH)], osems[c % 2])
    oh[nch - 2].wait()
    oh[nch - 1].wait()


_SC_KERNELS = None


def _get_sc_kernels():
    global _SC_KERNELS
    if _SC_KERNELS is None:
        mesh = plsc.VectorSubcoreMesh(core_axis_name="c", subcore_axis_name="s",
                                      num_cores=NC, num_subcores=NS)
        route = pl.kernel(
            _sc_route_body, mesh=mesh, compiler_params=_sc_params,
            out_type=jax.ShapeDtypeStruct((S, D_MODEL), jnp.float32),
            scratch_types=[
                pltpu.VMEM((S * E,), jnp.float32),
                pltpu.VMEM((CAP + 16,), jnp.int32),
                pltpu.VMEM((_RCH, D_MODEL), jnp.float32),
                pltpu.VMEM((_RCH, D_MODEL), jnp.float32),
                pltpu.SemaphoreType.DMA,
                pltpu.SemaphoreType.DMA,
                pltpu.SemaphoreType.DMA,
                pltpu.SemaphoreType.DMA,
            ],
        )
        combine = pl.kernel(
            _sc_combine_body, mesh=mesh, compiler_params=_sc_params,
            out_type=jax.ShapeDtypeStruct((S, D_MODEL), jnp.float32),
            scratch_types=[
                pltpu.VMEM((_CPW,), jnp.int32),
                pltpu.VMEM((_CPW,), jnp.int32),
                pltpu.VMEM((_CPW,), jnp.float32),
                pltpu.VMEM((_CPW,), jnp.float32),
                pltpu.VMEM((_CH, D_MODEL), jnp.float32),
                pltpu.VMEM((_CH, D_MODEL), jnp.float32),
                pltpu.VMEM((_CH, D_MODEL), jnp.float32),
                pltpu.VMEM((_CH, D_MODEL), jnp.float32),
                pltpu.VMEM((_CH, D_MODEL), jnp.float32),
                pltpu.VMEM((_CH, D_MODEL), jnp.float32),
                pltpu.SemaphoreType.DMA,
                pltpu.SemaphoreType.DMA,
                pltpu.SemaphoreType.DMA,
                pltpu.SemaphoreType.DMA,
                pltpu.SemaphoreType.DMA,
            ],
        )
        _SC_KERNELS = (route, combine)
    return _SC_KERNELS


def kernel(x, w_gate, W1, b1, W2, b2):
    ru = jax.random.uniform(jax.random.key(42), (S, E), dtype=jnp.float32) / 2.0
    sc_route, sc_combine = _get_sc_kernels()

    gates, ci0, cw0, ci1, cw1, loss = pl.pallas_call(
        _gating_body,
        out_shape=(
            jax.ShapeDtypeStruct((S, E), jnp.float32),
            jax.ShapeDtypeStruct((S, 1), jnp.int32),
            jax.ShapeDtypeStruct((S, 1), jnp.float32),
            jax.ShapeDtypeStruct((S, 1), jnp.int32),
            jax.ShapeDtypeStruct((S, 1), jnp.float32),
            jax.ShapeDtypeStruct((1, 1), jnp.float32),
        ),
        in_specs=[
            pl.BlockSpec((S, D_MODEL), lambda: (0, 0)),
            pl.BlockSpec((D_MODEL, E), lambda: (0, 0)),
            pl.BlockSpec((S, E), lambda: (0, 0)),
        ],
        out_specs=(
            pl.BlockSpec((S, E), lambda: (0, 0)),
            pl.BlockSpec((S, 1), lambda: (0, 0)),
            pl.BlockSpec((S, 1), lambda: (0, 0)),
            pl.BlockSpec((S, 1), lambda: (0, 0)),
            pl.BlockSpec((S, 1), lambda: (0, 0)),
            pl.BlockSpec((1, 1), lambda: (0, 0), memory_space=pltpu.SMEM),
        ),
    )(x, w_gate, ru)

    xd = sc_route(gates.reshape(S * E), x)

    o = pl.pallas_call(
        _ffn_body,
        grid=(E,),
        out_shape=jax.ShapeDtypeStruct((S, D_MODEL), jnp.float32),
        in_specs=[
            pl.BlockSpec((CAP, D_MODEL), lambda e: (e, 0)),
            pl.BlockSpec((1, D_MODEL, D_FF), lambda e: (e, 0, 0)),
            pl.BlockSpec((1, 1, D_FF), lambda e: (e, 0, 0)),
            pl.BlockSpec((1, D_FF, D_MODEL), lambda e: (e, 0, 0)),
            pl.BlockSpec((1, 1, D_MODEL), lambda e: (e, 0, 0)),
        ],
        out_specs=pl.BlockSpec((CAP, D_MODEL), lambda e: (e, 0)),
    )(xd, W1, b1.reshape(E, 1, D_FF), W2, b2.reshape(E, 1, D_MODEL))

    y = sc_combine(o, ci0.reshape(S), cw0.reshape(S),
                   ci1.reshape(S), cw1.reshape(S))

    return y, loss.reshape(())


# speedup vs baseline: 1.4196x; 1.3212x over previous
import functools

import jax
import jax.numpy as jnp
import numpy as np
from jax import lax
from jax.experimental import pallas as pl
from jax.experimental.pallas import tpu as pltpu
from jax.experimental.pallas import tpu_sc as plsc

D_MODEL = 1024
D_FF = 2048
E = 8
S = 2048
CAP = S // E
EPS = float(np.finfo(float).eps)
F32_INF_BITS = 0x7F800000

NC = 2
NS = 16
NW = NC * NS

_sc_params = pltpu.CompilerParams(needs_layout_passes=False)


def _gating_body(x_ref, wg_ref, ru_ref, gates_ref, ci0_ref, cw0_ref,
                 ci1_ref, cw1_ref, loss_ref):
    x = x_ref[...]
    logits = jnp.dot(x, wg_ref[...], preferred_element_type=jnp.float32)
    m = jnp.max(logits, axis=1, keepdims=True)
    ex = jnp.exp(logits - m)
    g = ex / jnp.sum(ex, axis=1, keepdims=True)

    cols = jax.lax.broadcasted_iota(jnp.int32, (S, E), 1)
    m1 = jnp.max(g, axis=1, keepdims=True)
    e1 = jnp.min(jnp.where(g == m1, cols, E), axis=1, keepdims=True)
    g_m = jnp.where(cols == e1, -jnp.inf, g)
    m2 = jnp.max(g_m, axis=1, keepdims=True)
    e2 = jnp.min(jnp.where(g_m == m2, cols, E), axis=1, keepdims=True)
    s12 = m1 + m2
    g1 = m1 / s12
    g2 = m2 / s12

    mean_g = jnp.mean(g, axis=0, keepdims=True)
    counts = jnp.sum((cols == e1).astype(jnp.float32), axis=0, keepdims=True)
    loss_ref[0, 0] = jnp.sum(counts / S * mean_g) / E * 0.1

    gcw1 = jnp.where(cols == e1, g1, 0.0)
    gcw2 = jnp.where(cols == e2, g2, 0.0)
    gcw2 = jnp.where(gcw2 > ru_ref[...], gcw2, 0.0)
    gcw = gcw1 + gcw2

    def bs_body(_, lohi):
        lo, hi = lohi
        mid = lo + (hi - lo) // 2
        t = jax.lax.bitcast_convert_type(mid, jnp.float32)
        cnt = jnp.sum((gcw > t).astype(jnp.int32), axis=0, keepdims=True)
        pred = cnt < CAP
        return jnp.where(pred, lo, mid + 1), jnp.where(pred, mid, hi)

    lo0 = jnp.zeros((1, E), jnp.int32)
    hi0 = jnp.full((1, E), F32_INF_BITS, jnp.int32)
    lo, _ = jax.lax.fori_loop(0, 31, bs_body, (lo0, hi0))
    t = jax.lax.bitcast_convert_type(lo, jnp.float32)

    gt = gcw > t
    n_gt = jnp.sum(gt.astype(jnp.int32), axis=0, keepdims=True)
    quota = (CAP - n_gt).astype(jnp.float32)
    eq = (gcw == t) & (gcw > 0.0)
    ri = jax.lax.broadcasted_iota(jnp.int32, (S, S), 0)
    ci = jax.lax.broadcasted_iota(jnp.int32, (S, S), 1)
    tri = (ci < ri).astype(jnp.float32)
    both = jnp.concatenate(
        [gt.astype(jnp.float32), eq.astype(jnp.float32)], axis=1)
    pref = jnp.dot(tri, both, preferred_element_type=jnp.float32)
    gt_prefix = pref[:, :E]
    eq_prefix = pref[:, E:]
    sel = gt | (eq & (eq_prefix < quota))
    gates_m = jnp.where(sel, gcw, 0.0)
    gates_ref[...] = gates_m

    slots = (gt_prefix + jnp.minimum(eq_prefix, quota)).astype(jnp.int32)
    flatpos = cols * CAP + slots
    ci0_ref[...] = jnp.sum(jnp.where((cols == e1) & sel, flatpos, 0),
                           axis=1, keepdims=True)
    cw0_ref[...] = jnp.sum(jnp.where(cols == e1, gates_m, 0.0),
                           axis=1, keepdims=True)
    ci1_ref[...] = jnp.sum(jnp.where((cols == e2) & sel, flatpos, 0),
                           axis=1, keepdims=True)
    cw1_ref[...] = jnp.sum(jnp.where(cols == e2, gates_m, 0.0),
                           axis=1, keepdims=True)


def _ffn_body(xd_ref, w1_ref, b1_ref, w2_ref, b2_ref,
              ci0_ref, cw0_ref, ci1_ref, cw1_ref, y_ref):
    e = pl.program_id(0)
    h = jnp.dot(xd_ref[...], w1_ref[0], preferred_element_type=jnp.float32)
    h = jnp.maximum(h + b1_ref[0], 0.0)
    o = (jnp.dot(h, w2_ref[0], preferred_element_type=jnp.float32)
         + b2_ref[0])

    half = S // 2
    for piece in range(2):
        rs = piece * half
        q = (jax.lax.broadcasted_iota(jnp.int32, (half, CAP), 1)
             + e * CAP)
        ci0 = ci0_ref[pl.ds(rs, half), :]
        cw0 = cw0_ref[pl.ds(rs, half), :]
        ci1 = ci1_ref[pl.ds(rs, half), :]
        cw1 = cw1_ref[pl.ds(rs, half), :]
        c_mat = (jnp.where(q == ci0, cw0, 0.0)
                 + jnp.where(q == ci1, cw1, 0.0))
        contrib = jnp.dot(c_mat, o, preferred_element_type=jnp.float32,
                          precision=jax.lax.Precision.HIGHEST)

        @pl.when(e == 0)
        def _():
            y_ref[pl.ds(rs, half), :] = contrib

        @pl.when(e > 0)
        def _():
            y_ref[pl.ds(rs, half), :] += contrib

    @pl.when(e == E - 1)
    def _():
        for piece in range(2):
            rs = piece * half
            yv = y_ref[pl.ds(rs, half), :]
            y_ref[pl.ds(rs, half), :] = jnp.where(yv == 0.0,
                                                  jnp.float32(EPS), yv)


_RP = 4
_RROWS = CAP // _RP
_RCH = 32


def _sc_route_body(gates_hbm, x_hbm, xd_hbm, gv, idxb,
                   xb0, xb1, gs0, gs1, os0, os1):
    wid = lax.axis_index("s") * NC + lax.axis_index("c")
    e = wid // _RP
    part = wid - e * _RP

    pltpu.sync_copy(gates_hbm, gv)
    z = jnp.zeros((16,), jnp.int32)
    for zz in range((CAP + 16) // 16):
        idxb[pl.ds(zz * 16, 16)] = z
    ecol = jnp.full((16,), e, jnp.int32)

    def step(i, cnt):
        rows = lax.iota(jnp.int32, 16) + i * 16
        vals = plsc.load_gather(gv, [rows * E + ecol])
        msk = vals > 0.0
        plsc.store_compressed(idxb.at[pl.ds(cnt, 16)], rows, mask=msk)
        return cnt + jnp.max(plsc.all_reduce_population_count(msk))

    lax.fori_loop(0, S // 16, step, jnp.int32(0))

    base = part * _RROWS
    xbufs = (xb0, xb1)
    gsems = (gs0, gs1)
    osems = (os0, os1)
    nch = _RROWS // _RCH
    gh = [None] * nch
    oh = [None] * nch
    gh[0] = pltpu.async_copy(
        x_hbm.at[idxb.at[pl.ds(base, _RCH)]], xbufs[0], gsems[0])
    for c in range(nch):
        if c + 1 < nch:
            gh[c + 1] = pltpu.async_copy(
                x_hbm.at[idxb.at[pl.ds(base + (c + 1) * _RCH, _RCH)]],
                xbufs[(c + 1) % 2], gsems[(c + 1) % 2])
        gh[c].wait()
        oh[c] = pltpu.async_copy(
            xbufs[c % 2],
            xd_hbm.at[pl.ds(e * CAP + base + c * _RCH, _RCH)],
            osems[c % 2])
    for c in range(nch):
        oh[c].wait()


_SC_KERNELS = None


def _get_sc_kernels():
    global _SC_KERNELS
    if _SC_KERNELS is None:
        mesh = plsc.VectorSubcoreMesh(core_axis_name="c", subcore_axis_name="s",
                                      num_cores=NC, num_subcores=NS)
        route = pl.kernel(
            _sc_route_body, mesh=mesh, compiler_params=_sc_params,
            out_type=jax.ShapeDtypeStruct((S, D_MODEL), jnp.float32),
            scratch_types=[
                pltpu.VMEM((S * E,), jnp.float32),
                pltpu.VMEM((CAP + 16,), jnp.int32),
                pltpu.VMEM((_RCH, D_MODEL), jnp.float32),
                pltpu.VMEM((_RCH, D_MODEL), jnp.float32),
                pltpu.SemaphoreType.DMA,
                pltpu.SemaphoreType.DMA,
                pltpu.SemaphoreType.DMA,
                pltpu.SemaphoreType.DMA,
            ],
        )
        _SC_KERNELS = route
    return _SC_KERNELS


def kernel(x, w_gate, W1, b1, W2, b2):
    ru = jax.random.uniform(jax.random.key(42), (S, E), dtype=jnp.float32) / 2.0
    sc_route = _get_sc_kernels()

    gates, ci0, cw0, ci1, cw1, loss = pl.pallas_call(
        _gating_body,
        out_shape=(
            jax.ShapeDtypeStruct((S, E), jnp.float32),
            jax.ShapeDtypeStruct((S, 1), jnp.int32),
            jax.ShapeDtypeStruct((S, 1), jnp.float32),
            jax.ShapeDtypeStruct((S, 1), jnp.int32),
            jax.ShapeDtypeStruct((S, 1), jnp.float32),
            jax.ShapeDtypeStruct((1, 1), jnp.float32),
        ),
        in_specs=[
            pl.BlockSpec((S, D_MODEL), lambda: (0, 0)),
            pl.BlockSpec((D_MODEL, E), lambda: (0, 0)),
            pl.BlockSpec((S, E), lambda: (0, 0)),
        ],
        out_specs=(
            pl.BlockSpec((S, E), lambda: (0, 0)),
            pl.BlockSpec((S, 1), lambda: (0, 0)),
            pl.BlockSpec((S, 1), lambda: (0, 0)),
            pl.BlockSpec((S, 1), lambda: (0, 0)),
            pl.BlockSpec((S, 1), lambda: (0, 0)),
            pl.BlockSpec((1, 1), lambda: (0, 0), memory_space=pltpu.SMEM),
        ),
    )(x, w_gate, ru)

    xd = sc_route(gates.reshape(S * E), x)

    y = pl.pallas_call(
        _ffn_body,
        grid=(E,),
        out_shape=jax.ShapeDtypeStruct((S, D_MODEL), jnp.float32),
        in_specs=[
            pl.BlockSpec((CAP, D_MODEL), lambda e: (e, 0)),
            pl.BlockSpec((1, D_MODEL, D_FF), lambda e: (e, 0, 0)),
            pl.BlockSpec((1, 1, D_FF), lambda e: (e, 0, 0)),
            pl.BlockSpec((1, D_FF, D_MODEL), lambda e: (e, 0, 0)),
            pl.BlockSpec((1, 1, D_MODEL), lambda e: (e, 0, 0)),
            pl.BlockSpec((S, 1), lambda e: (0, 0)),
            pl.BlockSpec((S, 1), lambda e: (0, 0)),
            pl.BlockSpec((S, 1), lambda e: (0, 0)),
            pl.BlockSpec((S, 1), lambda e: (0, 0)),
        ],
        out_specs=pl.BlockSpec((S, D_MODEL), lambda e: (0, 0)),
    )(xd, W1, b1.reshape(E, 1, D_FF), W2, b2.reshape(E, 1, D_MODEL),
      ci0, cw0, ci1, cw1)

    return y, loss.reshape(())


# speedup vs baseline: 1.9679x; 1.3862x over previous
import functools

import jax
import jax.numpy as jnp
import numpy as np
from jax import lax
from jax.experimental import pallas as pl
from jax.experimental.pallas import tpu as pltpu
from jax.experimental.pallas import tpu_sc as plsc

D_MODEL = 1024
D_FF = 2048
E = 8
S = 2048
CAP = S // E
EPS = float(np.finfo(float).eps)
F32_INF_BITS = 0x7F800000

NC = 2
NS = 16
NW = NC * NS

_sc_params = pltpu.CompilerParams(needs_layout_passes=False)


def _gating_body(x_ref, wg_ref, ru_ref, gates_ref, ci0_ref, cw0_ref,
                 ci1_ref, cw1_ref, loss_ref):
    x = x_ref[...]
    logits = jnp.dot(x, wg_ref[...], preferred_element_type=jnp.float32)
    m = jnp.max(logits, axis=1, keepdims=True)
    ex = jnp.exp(logits - m)
    g = ex / jnp.sum(ex, axis=1, keepdims=True)

    cols = jax.lax.broadcasted_iota(jnp.int32, (S, E), 1)
    m1 = jnp.max(g, axis=1, keepdims=True)
    e1 = jnp.min(jnp.where(g == m1, cols, E), axis=1, keepdims=True)
    g_m = jnp.where(cols == e1, -jnp.inf, g)
    m2 = jnp.max(g_m, axis=1, keepdims=True)
    e2 = jnp.min(jnp.where(g_m == m2, cols, E), axis=1, keepdims=True)
    s12 = m1 + m2
    g1 = m1 / s12
    g2 = m2 / s12

    mean_g = jnp.mean(g, axis=0, keepdims=True)
    counts = jnp.sum((cols == e1).astype(jnp.float32), axis=0, keepdims=True)
    loss_ref[0, 0] = jnp.sum(counts / S * mean_g) / E * 0.1

    gcw1 = jnp.where(cols == e1, g1, 0.0)
    gcw2 = jnp.where(cols == e2, g2, 0.0)
    gcw2 = jnp.where(gcw2 > ru_ref[...], gcw2, 0.0)
    gcw = gcw1 + gcw2

    def bs_body(_, lohi):
        lo, hi = lohi
        mid = lo + (hi - lo) // 2
        t = jax.lax.bitcast_convert_type(mid, jnp.float32)
        cnt = jnp.sum((gcw > t).astype(jnp.int32), axis=0, keepdims=True)
        pred = cnt < CAP
        return jnp.where(pred, lo, mid + 1), jnp.where(pred, mid, hi)

    lo0 = jnp.zeros((1, E), jnp.int32)
    hi0 = jnp.full((1, E), F32_INF_BITS, jnp.int32)
    lo, _ = jax.lax.fori_loop(0, 31, bs_body, (lo0, hi0))
    t = jax.lax.bitcast_convert_type(lo, jnp.float32)

    gt = gcw > t
    n_gt = jnp.sum(gt.astype(jnp.int32), axis=0, keepdims=True)
    quota = (CAP - n_gt).astype(jnp.float32)
    eq = (gcw == t) & (gcw > 0.0)
    ri = jax.lax.broadcasted_iota(jnp.int32, (S, S), 0)
    ci = jax.lax.broadcasted_iota(jnp.int32, (S, S), 1)
    tri = (ci < ri).astype(jnp.float32)
    both = jnp.concatenate(
        [gt.astype(jnp.float32), eq.astype(jnp.float32)], axis=1)
    pref = jnp.dot(tri, both, preferred_element_type=jnp.float32)
    gt_prefix = pref[:, :E]
    eq_prefix = pref[:, E:]
    sel = gt | (eq & (eq_prefix < quota))
    gates_m = jnp.where(sel, gcw, 0.0)
    gates_ref[...] = gates_m

    slots = (gt_prefix + jnp.minimum(eq_prefix, quota)).astype(jnp.int32)
    flatpos = cols * CAP + slots
    ci0_ref[...] = jnp.sum(jnp.where((cols == e1) & sel, flatpos, 0),
                           axis=1, keepdims=True)
    cw0_ref[...] = jnp.sum(jnp.where(cols == e1, gates_m, 0.0),
                           axis=1, keepdims=True)
    ci1_ref[...] = jnp.sum(jnp.where((cols == e2) & sel, flatpos, 0),
                           axis=1, keepdims=True)
    cw1_ref[...] = jnp.sum(jnp.where(cols == e2, gates_m, 0.0),
                           axis=1, keepdims=True)


def _ffn_body(xd_ref, w1_ref, b1_ref, w2_ref, b2_ref,
              ci0_ref, cw0_ref, ci1_ref, cw1_ref, y_ref):
    e = pl.program_id(0)
    h = jnp.dot(xd_ref[...], w1_ref[0], preferred_element_type=jnp.float32)
    h = jnp.maximum(h + b1_ref[0], 0.0)
    o = (jnp.dot(h, w2_ref[0], preferred_element_type=jnp.float32)
         + b2_ref[0])

    half = S // 2
    for piece in range(2):
        rs = piece * half
        q = (jax.lax.broadcasted_iota(jnp.int32, (half, CAP), 1)
             + e * CAP)
        ci0 = ci0_ref[pl.ds(rs, half), :]
        cw0 = cw0_ref[pl.ds(rs, half), :]
        ci1 = ci1_ref[pl.ds(rs, half), :]
        cw1 = cw1_ref[pl.ds(rs, half), :]
        c_mat = (jnp.where(q == ci0, cw0, 0.0)
                 + jnp.where(q == ci1, cw1, 0.0))
        contrib = jnp.dot(c_mat, o, preferred_element_type=jnp.float32)

        @pl.when(e == 0)
        def _():
            y_ref[pl.ds(rs, half), :] = contrib

        @pl.when(e > 0)
        def _():
            y_ref[pl.ds(rs, half), :] += contrib

    @pl.when(e == E - 1)
    def _():
        for piece in range(2):
            rs = piece * half
            yv = y_ref[pl.ds(rs, half), :]
            y_ref[pl.ds(rs, half), :] = jnp.where(yv == 0.0,
                                                  jnp.float32(EPS), yv)


_RP = 4
_RROWS = CAP // _RP
_RCH = 32


def _sc_route_body(gates_hbm, x_hbm, xd_hbm, gv, idxb,
                   xb0, xb1, gs0, gs1, os0, os1):
    wid = lax.axis_index("s") * NC + lax.axis_index("c")
    e = wid // _RP
    part = wid - e * _RP

    pltpu.sync_copy(gates_hbm, gv)
    z = jnp.zeros((16,), jnp.int32)
    for zz in range((CAP + 16) // 16):
        idxb[pl.ds(zz * 16, 16)] = z
    ecol = jnp.full((16,), e, jnp.int32)

    def step(i, cnt):
        rows = lax.iota(jnp.int32, 16) + i * 16
        vals = plsc.load_gather(gv, [rows * E + ecol])
        msk = vals > 0.0
        plsc.store_compressed(idxb.at[pl.ds(cnt, 16)], rows, mask=msk)
        return cnt + jnp.max(plsc.all_reduce_population_count(msk))

    lax.fori_loop(0, S // 16, step, jnp.int32(0))

    base = part * _RROWS
    xbufs = (xb0, xb1)
    gsems = (gs0, gs1)
    osems = (os0, os1)
    nch = _RROWS // _RCH
    gh = [None] * nch
    oh = [None] * nch
    gh[0] = pltpu.async_copy(
        x_hbm.at[idxb.at[pl.ds(base, _RCH)]], xbufs[0], gsems[0])
    for c in range(nch):
        if c + 1 < nch:
            gh[c + 1] = pltpu.async_copy(
                x_hbm.at[idxb.at[pl.ds(base + (c + 1) * _RCH, _RCH)]],
                xbufs[(c + 1) % 2], gsems[(c + 1) % 2])
        gh[c].wait()
        oh[c] = pltpu.async_copy(
            xbufs[c % 2],
            xd_hbm.at[pl.ds(e * CAP + base + c * _RCH, _RCH)],
            osems[c % 2])
    for c in range(nch):
        oh[c].wait()


_SC_KERNELS = None
_RU = None


def _get_ru():
    global _RU
    if _RU is not None:
        return jnp.asarray(_RU)
    try:
        with jax.ensure_compile_time_eval():
            val = (jax.random.uniform(jax.random.key(42), (S, E),
                                      dtype=jnp.float32) / 2.0)
        _RU = np.asarray(val)
        return jnp.asarray(_RU)
    except Exception:
        return (jax.random.uniform(jax.random.key(42), (S, E),
                                   dtype=jnp.float32) / 2.0)


def _get_sc_kernels():
    global _SC_KERNELS
    if _SC_KERNELS is None:
        mesh = plsc.VectorSubcoreMesh(core_axis_name="c", subcore_axis_name="s",
                                      num_cores=NC, num_subcores=NS)
        route = pl.kernel(
            _sc_route_body, mesh=mesh, compiler_params=_sc_params,
            out_type=jax.ShapeDtypeStruct((S, D_MODEL), jnp.float32),
            scratch_types=[
                pltpu.VMEM((S * E,), jnp.float32),
                pltpu.VMEM((CAP + 16,), jnp.int32),
                pltpu.VMEM((_RCH, D_MODEL), jnp.float32),
                pltpu.VMEM((_RCH, D_MODEL), jnp.float32),
                pltpu.SemaphoreType.DMA,
                pltpu.SemaphoreType.DMA,
                pltpu.SemaphoreType.DMA,
                pltpu.SemaphoreType.DMA,
            ],
        )
        _SC_KERNELS = route
    return _SC_KERNELS


def kernel(x, w_gate, W1, b1, W2, b2):
    ru = _get_ru()
    sc_route = _get_sc_kernels()

    gates, ci0, cw0, ci1, cw1, loss = pl.pallas_call(
        _gating_body,
        out_shape=(
            jax.ShapeDtypeStruct((S, E), jnp.float32),
            jax.ShapeDtypeStruct((S, 1), jnp.int32),
            jax.ShapeDtypeStruct((S, 1), jnp.float32),
            jax.ShapeDtypeStruct((S, 1), jnp.int32),
            jax.ShapeDtypeStruct((S, 1), jnp.float32),
            jax.ShapeDtypeStruct((1, 1), jnp.float32),
        ),
        in_specs=[
            pl.BlockSpec((S, D_MODEL), lambda: (0, 0)),
            pl.BlockSpec((D_MODEL, E), lambda: (0, 0)),
            pl.BlockSpec((S, E), lambda: (0, 0)),
        ],
        out_specs=(
            pl.BlockSpec((S, E), lambda: (0, 0)),
            pl.BlockSpec((S, 1), lambda: (0, 0)),
            pl.BlockSpec((S, 1), lambda: (0, 0)),
            pl.BlockSpec((S, 1), lambda: (0, 0)),
            pl.BlockSpec((S, 1), lambda: (0, 0)),
            pl.BlockSpec((1, 1), lambda: (0, 0), memory_space=pltpu.SMEM),
        ),
    )(x, w_gate, ru)

    xd = sc_route(gates.reshape(S * E), x)

    y = pl.pallas_call(
        _ffn_body,
        grid=(E,),
        out_shape=jax.ShapeDtypeStruct((S, D_MODEL), jnp.float32),
        in_specs=[
            pl.BlockSpec((CAP, D_MODEL), lambda e: (e, 0)),
            pl.BlockSpec((1, D_MODEL, D_FF), lambda e: (e, 0, 0)),
            pl.BlockSpec((1, 1, D_FF), lambda e: (e, 0, 0)),
            pl.BlockSpec((1, D_FF, D_MODEL), lambda e: (e, 0, 0)),
            pl.BlockSpec((1, 1, D_MODEL), lambda e: (e, 0, 0)),
            pl.BlockSpec((S, 1), lambda e: (0, 0)),
            pl.BlockSpec((S, 1), lambda e: (0, 0)),
            pl.BlockSpec((S, 1), lambda e: (0, 0)),
            pl.BlockSpec((S, 1), lambda e: (0, 0)),
        ],
        out_specs=pl.BlockSpec((S, D_MODEL), lambda e: (0, 0)),
    )(xd, W1, b1.reshape(E, 1, D_FF), W2, b2.reshape(E, 1, D_MODEL),
      ci0, cw0, ci1, cw1)

    return y, loss.reshape(())
